# baseline (speedup 1.0000x reference)
import functools

import jax
import jax.numpy as jnp
from jax import lax
from jax.experimental import pallas as pl
from jax.experimental.pallas import tpu as pltpu
from jax.experimental.pallas import tpu_sc as plsc

N = 10000
E = 320000
DIN = 128
D = 256
H = 16
DH = 16
FF = 512
L = 4
DOUT = 128


def _stats(z):
    m = jnp.mean(z, axis=0)
    v = jnp.mean((z - m) ** 2, axis=0)
    return m, v


def _bnorm(z, m, v, g, b):
    return (z - m) / jnp.sqrt(v + 1e-5) * g + b


def _embed_body(x_ref, w1_ref, b1_ref, g_ref, bb_ref, w2_ref, b2_ref, h_ref):
    z = jnp.dot(x_ref[...], w1_ref[...], preferred_element_type=jnp.float32) + b1_ref[...]
    m, v = _stats(z)
    h1 = jnp.maximum(_bnorm(z, m, v, g_ref[...], bb_ref[...]), 0.0)
    h_ref[...] = jnp.dot(h1, w2_ref[...], preferred_element_type=jnp.float32) + b2_ref[...] + h1


def _attn_body(h_ref, w_ref, a_ref, b_ref, hh_ref, el_ref, er_ref):
    hh = jnp.dot(h_ref[...], w_ref[...], preferred_element_type=jnp.float32)
    hh_ref[...] = hh
    el_ref[...] = jnp.dot(hh, a_ref[...], preferred_element_type=jnp.float32)
    er_ref[...] = jnp.dot(hh, b_ref[...], preferred_element_type=jnp.float32)


def _post1_body(h_ref, r0_ref, r1_ref, den_ref, e16_ref, gb_ref, g1_ref, b1_ref, out_ref):
    rstu = jnp.concatenate([r0_ref[...], r1_ref[...]], axis=1)
    rden = 1.0 / jnp.maximum(den_ref[...], 1e-30)
    scale = jnp.dot(rden, e16_ref[...], preferred_element_type=jnp.float32)
    g = h_ref[...] + rstu * scale + gb_ref[...]
    m, v = _stats(g)
    out_ref[...] = _bnorm(g, m, v, g1_ref[...], b1_ref[...])


def _post2_body(t_ref, fw1_ref, fb1_ref, fw2_ref, fb2_ref, g2_ref, b2_ref, out_ref):
    t = t_ref[...]
    y = jnp.maximum(
        jnp.dot(t, fw1_ref[...], preferred_element_type=jnp.float32) + fb1_ref[...], 0.0)
    y = jnp.dot(y, fw2_ref[...], preferred_element_type=jnp.float32) + fb2_ref[...]
    t2 = t + y
    m2, v2 = _stats(t2)
    out_ref[...] = _bnorm(t2, m2, v2, g2_ref[...], b2_ref[...])


def _dec_body(h_ref, w1_ref, b1_ref, g_ref, bb_ref, w2_ref, b2_ref, out_ref):
    z = jnp.dot(h_ref[...], w1_ref[...], preferred_element_type=jnp.float32) + b1_ref[...]
    m, v = _stats(z)
    d1 = jnp.maximum(_bnorm(z, m, v, g_ref[...], bb_ref[...]), 0.0)
    out_ref[...] = jnp.dot(d1, w2_ref[...], preferred_element_type=jnp.float32) + b2_ref[...]


def _f32(shape):
    return jax.ShapeDtypeStruct(shape, jnp.float32)


CHUNK = 80
NTILE = 16
NPAD = 10112
CPT = 250
EPAD = NTILE * CPT * CHUNK
CROWS = EPAD // CHUNK
RPT = NPAD // NTILE
DW = 144


def _edge_body(src_h, dst_h, hhx_h, er_h, rstu_h,
               acc, sbuf, dbuf, ibuf, erb, hhb, prodb, gsem, isem, scsem):
    c = lax.axis_index("c")
    s = lax.axis_index("s")

    def zrow(i, carry):
        for k in range(DW // 16):
            prodb[0, i, pl.ds(k * 16, 16)] = jnp.zeros((16,), jnp.float32)
        return carry

    lax.fori_loop(0, CHUNK, zrow, 0)
    rbase = pl.multiple_of(s * RPT, 8)
    off = 0
    for sz in (80, 80, 80, 80, 80, 80, 80, RPT - 7 * 80):
        pltpu.sync_copy(prodb.at[0, pl.ds(0, sz)], acc.at[pl.ds(rbase + off, sz)])
        off += sz
    plsc.subcore_barrier()

    def idx_copy(chunk_id, i3):
        row = s * CPT + chunk_id
        pltpu.async_copy(src_h.at[row], sbuf.at[i3], isem.at[i3])
        pltpu.async_copy(dst_h.at[row], dbuf.at[i3], isem.at[i3])

    def idx_wait(i3):
        pltpu.make_async_copy(src_h.at[0], sbuf.at[i3], isem.at[i3]).wait()
        pltpu.make_async_copy(dst_h.at[0], dbuf.at[i3], isem.at[i3]).wait()

    def ibuf_compute(i3):
        for t in range(CHUNK // 16):
            sl = pl.ds(t * 16, 16)
            ibuf[i3, sl] = sbuf[i3, sl] + c * N

    def gathers_issue(s2, i3):
        pltpu.async_copy(hhx_h.at[ibuf.at[i3]], hhb.at[s2], gsem.at[s2])
        pltpu.async_copy(er_h.at[dbuf.at[i3]], erb.at[s2], gsem.at[s2])

    def gathers_wait(s2, i3):
        pltpu.make_async_copy(hhx_h.at[ibuf.at[i3]], hhb.at[s2], gsem.at[s2]).wait()
        pltpu.make_async_copy(er_h.at[dbuf.at[i3]], erb.at[s2], gsem.at[s2]).wait()

    def scatter_issue(s2, i3):
        pltpu.async_copy(prodb.at[s2], acc.at[dbuf.at[i3]], scsem.at[s2], add=True)

    def scatter_wait(s2, i3):
        pltpu.make_async_copy(prodb.at[s2], acc.at[dbuf.at[i3]], scsem.at[s2]).wait()

    def _bcast(wv, lane_idx):
        lane = jnp.full((16, 1), lane_idx, jnp.int32)
        return lax.gather(
            wv, lane,
            lax.GatherDimensionNumbers(
                offset_dims=(), collapsed_slice_dims=(0,),
                start_index_map=(0,)),
            slice_sizes=(1,),
            mode=lax.GatherScatterMode.PROMISE_IN_BOUNDS)

    def compute_chunk(s2):
        def ebody(j, cy):
            elv = plsc.bitcast(hhb[s2, j, pl.ds(128, 32)], jnp.float32)
            erv = erb[s2, j]
            ev = elv + erv
            ev = jnp.where(ev > 0, ev, 0.2 * ev)
            wv = jnp.exp(ev)
            prodb[s2, j, pl.ds(128, 16)] = wv
            for k in range(4):
                pair = hhb[s2, j, pl.ds(k * 32, 32)]
                va, vb = plsc.unpack(pair, format=plsc.PackFormat.INTERLEAVED)
                wa = _bcast(wv, c * 8 + 2 * k)
                wb = _bcast(wv, c * 8 + 2 * k + 1)
                prodb[s2, j, pl.ds(2 * k * 16, 16)] = va * wa
                prodb[s2, j, pl.ds((2 * k + 1) * 16, 16)] = vb * wb
            return cy

        lax.fori_loop(0, CHUNK, ebody, 0, unroll=4)

    row0 = s * CPT
    pltpu.sync_copy(src_h.at[row0], sbuf.at[0])
    pltpu.sync_copy(dst_h.at[row0], dbuf.at[0])
    ibuf_compute(0)
    gathers_issue(0, 0)
    idx_copy(1, 1)

    def chunk_iter(ci, carry):
        s0 = lax.rem(ci, 2)
        s1 = 1 - s0
        i0 = lax.rem(ci, 3)
        i1 = lax.rem(ci + 1, 3)
        i2 = lax.rem(ci + 2, 3)

        @pl.when(ci + 1 < CPT)
        def _():
            idx_wait(i1)
            ibuf_compute(i1)

            @pl.when(ci >= 1)
            def _():
                scatter_wait(s1, i2)

            gathers_issue(s1, i1)

        @pl.when(ci + 2 < CPT)
        def _():
            idx_copy(ci + 2, i2)

        gathers_wait(s0, i0)
        compute_chunk(s0)
        scatter_issue(s0, i0)
        return carry

    lax.fori_loop(0, CPT, chunk_iter, 0)
    scatter_wait((CPT - 2) % 2, (CPT - 2) % 3)
    scatter_wait((CPT - 1) % 2, (CPT - 1) % 3)
    plsc.subcore_barrier()
    pltpu.sync_copy(acc.at[pl.ds(rbase, RPT)], rstu_h.at[c, pl.ds(rbase, RPT)])


@functools.lru_cache(maxsize=1)
def _make_edge_pass():
    return functools.partial(
        pl.kernel,
        mesh=plsc.VectorSubcoreMesh(core_axis_name="c", subcore_axis_name="s"),
        compiler_params=pltpu.CompilerParams(
            use_tc_tiling_on_sc=False, needs_layout_passes=False),
        out_type=jax.ShapeDtypeStruct((2, NPAD, DW), jnp.float32),
        scratch_types=[
            pltpu.VMEM_SHARED((NPAD, DW), jnp.float32),
            pltpu.VMEM((3, CHUNK), jnp.int32),
            pltpu.VMEM((3, CHUNK), jnp.int32),
            pltpu.VMEM((3, CHUNK), jnp.int32),
            pltpu.VMEM((2, CHUNK, H), jnp.float32),
            pltpu.VMEM((2, CHUNK, 160), jnp.bfloat16),
            pltpu.VMEM((2, CHUNK, DW), jnp.float32),
            pltpu.SemaphoreType.DMA((2,)),
            pltpu.SemaphoreType.DMA((3,)),
            pltpu.SemaphoreType.DMA((2,)),
        ],
    )(_edge_body)


def _edge_pass(*args):
    return _make_edge_pass()(*args)


def kernel(x, edge_index, emb_W1, emb_b1, emb_bn_g, emb_bn_b, emb_W2, emb_b2,
           gat_W, attn_l, attn_r, gat_bias, bn1_g, bn1_b, ff_W1, ff_b1,
           ff_W2, ff_b2, bn2_g, bn2_b, dec_W1, dec_b1, dec_bn_g, dec_bn_b,
           dec_W2, dec_b2):
    src = edge_index[0]
    dst = edge_index[1]
    src_p = src.reshape(CROWS, CHUNK)
    dst_p = dst.reshape(CROWS, CHUNK)
    r = lambda a: a.reshape(1, -1)

    eyeh = jnp.eye(H, dtype=jnp.float32)
    abig = (attn_l[:, :, :, None] * eyeh[None, :, None, :]).reshape(L, D, H)
    bbig = (attn_r[:, :, :, None] * eyeh[None, :, None, :]).reshape(L, D, H)
    e16 = jnp.kron(eyeh, jnp.ones((1, DH), jnp.float32))

    h = pl.pallas_call(_embed_body, out_shape=_f32((N, D)))(
        x, emb_W1, r(emb_b1), r(emb_bn_g), r(emb_bn_b), emb_W2, r(emb_b2))

    for l in range(L):
        hh, el, er = pl.pallas_call(
            _attn_body, out_shape=[_f32((N, D)), _f32((N, H)), _f32((N, H))],
        )(h, gat_W[l], abig[l], bbig[l])

        el_bits = lax.bitcast_convert_type(el, jnp.bfloat16).reshape(N, 2 * H)
        halves = []
        for cc in range(2):
            hp = hh[:, cc * 128:(cc + 1) * 128].astype(jnp.bfloat16)
            hp = hp.reshape(N, 4, 2, 16).transpose(0, 1, 3, 2).reshape(N, 128)
            halves.append(jnp.concatenate([hp, el_bits], axis=1))
        hhx_cat = jnp.concatenate(halves, axis=0)
        er_p = jnp.pad(er, ((0, NPAD - N), (0, 0)))
        rstu2 = _edge_pass(src_p, dst_p, hhx_cat, er_p)

        t = pl.pallas_call(
            _post1_body, out_shape=_f32((N, D)),
        )(h, rstu2[0, :N, :128], rstu2[1, :N, :128], rstu2[0, :N, 128:], e16,
          r(gat_bias[l]), r(bn1_g[l]), r(bn1_b[l]))
        h = pl.pallas_call(
            _post2_body, out_shape=_f32((N, D)),
        )(t, ff_W1[l], r(ff_b1[l]), ff_W2[l], r(ff_b2[l]), r(bn2_g[l]), r(bn2_b[l]))

    out = pl.pallas_call(_dec_body, out_shape=_f32((N, DOUT)))(
        h, dec_W1, r(dec_b1), r(dec_bn_g), r(dec_bn_b), dec_W2, r(dec_b2))
    return out


# speedup vs baseline: 1.6553x; 1.6553x over previous
import functools

import jax
import jax.numpy as jnp
from jax import lax
from jax.experimental import pallas as pl
from jax.experimental.pallas import tpu as pltpu
from jax.experimental.pallas import tpu_sc as plsc

N = 10000
E = 320000
DIN = 128
D = 256
H = 16
DH = 16
FF = 512
L = 4
DOUT = 128


def _stats(z):
    m = jnp.mean(z, axis=0)
    v = jnp.mean((z - m) ** 2, axis=0)
    return m, v


def _bnorm(z, m, v, g, b):
    return (z - m) / jnp.sqrt(v + 1e-5) * g + b


def _embed_body(x_ref, w1_ref, b1_ref, g_ref, bb_ref, w2_ref, b2_ref, h_ref):
    z = jnp.dot(x_ref[...], w1_ref[...], preferred_element_type=jnp.float32) + b1_ref[...]
    m, v = _stats(z)
    h1 = jnp.maximum(_bnorm(z, m, v, g_ref[...], bb_ref[...]), 0.0)
    h_ref[...] = jnp.dot(h1, w2_ref[...], preferred_element_type=jnp.float32) + b2_ref[...] + h1


def _attn_body(h_ref, w_ref, a_ref, b_ref, hhx_ref, er_ref):
    hh = jnp.dot(h_ref[...], w_ref[...], preferred_element_type=jnp.float32)
    el = jnp.dot(hh, a_ref[...], preferred_element_type=jnp.float32)
    hhx_ref[0] = jnp.concatenate([hh[:, :128], el], axis=1)
    hhx_ref[1] = jnp.concatenate([hh[:, 128:], el], axis=1)
    er_ref[...] = jnp.dot(hh, b_ref[...], preferred_element_type=jnp.float32)


def _post1_body(h_ref, r0_ref, r1_ref, den_ref, e16_ref, gb_ref, g1_ref, b1_ref, out_ref):
    rstu = jnp.concatenate([r0_ref[...], r1_ref[...]], axis=1)
    rden = 1.0 / jnp.maximum(den_ref[...], 1e-30)
    scale = jnp.dot(rden, e16_ref[...], preferred_element_type=jnp.float32)
    g = h_ref[...] + rstu * scale + gb_ref[...]
    m, v = _stats(g)
    out_ref[...] = _bnorm(g, m, v, g1_ref[...], b1_ref[...])


def _post2_body(t_ref, fw1_ref, fb1_ref, fw2_ref, fb2_ref, g2_ref, b2_ref, out_ref):
    t = t_ref[...]
    y = jnp.maximum(
        jnp.dot(t, fw1_ref[...], preferred_element_type=jnp.float32) + fb1_ref[...], 0.0)
    y = jnp.dot(y, fw2_ref[...], preferred_element_type=jnp.float32) + fb2_ref[...]
    t2 = t + y
    m2, v2 = _stats(t2)
    out_ref[...] = _bnorm(t2, m2, v2, g2_ref[...], b2_ref[...])


def _dec_body(h_ref, w1_ref, b1_ref, g_ref, bb_ref, w2_ref, b2_ref, out_ref):
    z = jnp.dot(h_ref[...], w1_ref[...], preferred_element_type=jnp.float32) + b1_ref[...]
    m, v = _stats(z)
    d1 = jnp.maximum(_bnorm(z, m, v, g_ref[...], bb_ref[...]), 0.0)
    out_ref[...] = jnp.dot(d1, w2_ref[...], preferred_element_type=jnp.float32) + b2_ref[...]


def _f32(shape):
    return jax.ShapeDtypeStruct(shape, jnp.float32)


CHUNK = 80
NTILE = 16
NPAD = 10112
CPT = 250
EPAD = NTILE * CPT * CHUNK
CROWS = EPAD // CHUNK
RPT = NPAD // NTILE
DW = 144


def _edge_body(src_h, dst_h, hhx_h, er_h, rstu_h,
               acc, sbuf, dbuf, ibuf, erb, hhb, gsem, isem, scsem):
    c = lax.axis_index("c")
    s = lax.axis_index("s")

    def zrow(i, carry):
        for k in range(DW // 16):
            hhb[0, i, pl.ds(k * 16, 16)] = jnp.zeros((16,), jnp.float32)
        return carry

    lax.fori_loop(0, CHUNK, zrow, 0)
    rbase = pl.multiple_of(s * RPT, 8)
    off = 0
    for sz in (80, 80, 80, 80, 80, 80, 80, RPT - 7 * 80):
        pltpu.sync_copy(hhb.at[0, pl.ds(0, sz)], acc.at[pl.ds(rbase + off, sz)])
        off += sz
    plsc.subcore_barrier()

    def idx_copy(chunk_id, i3):
        row = s * CPT + chunk_id
        pltpu.async_copy(src_h.at[row], sbuf.at[i3], isem.at[i3])
        pltpu.async_copy(dst_h.at[row], dbuf.at[i3], isem.at[i3])

    def idx_wait(i3):
        pltpu.make_async_copy(src_h.at[0], sbuf.at[i3], isem.at[i3]).wait()
        pltpu.make_async_copy(dst_h.at[0], dbuf.at[i3], isem.at[i3]).wait()

    def ibuf_compute(i3):
        for t in range(CHUNK // 16):
            sl = pl.ds(t * 16, 16)
            ibuf[i3, sl] = sbuf[i3, sl] + c * N

    def gathers_issue(s2, i3):
        pltpu.async_copy(hhx_h.at[ibuf.at[i3]], hhb.at[s2], gsem.at[s2])
        pltpu.async_copy(er_h.at[dbuf.at[i3]], erb.at[s2], gsem.at[s2])

    def gathers_wait(s2, i3):
        pltpu.make_async_copy(hhx_h.at[ibuf.at[i3]], hhb.at[s2], gsem.at[s2]).wait()
        pltpu.make_async_copy(er_h.at[dbuf.at[i3]], erb.at[s2], gsem.at[s2]).wait()

    def scatter_issue(s2, i3):
        pltpu.async_copy(hhb.at[s2], acc.at[dbuf.at[i3]], scsem.at[s2], add=True)

    def scatter_wait(s2, i3):
        pltpu.make_async_copy(hhb.at[s2], acc.at[dbuf.at[i3]], scsem.at[s2]).wait()

    def _bcast(wv, lane_idx):
        lane = jnp.full((16, 1), lane_idx, jnp.int32)
        return lax.gather(
            wv, lane,
            lax.GatherDimensionNumbers(
                offset_dims=(), collapsed_slice_dims=(0,),
                start_index_map=(0,)),
            slice_sizes=(1,),
            mode=lax.GatherScatterMode.PROMISE_IN_BOUNDS)

    def compute_chunk(s2):
        def ebody(j, cy):
            elv = hhb[s2, j, pl.ds(128, 16)]
            erv = erb[s2, j]
            ev = elv + erv
            ev = jnp.where(ev > 0, ev, 0.2 * ev)
            wv = jnp.exp(ev)
            hhb[s2, j, pl.ds(128, 16)] = wv
            for hloc in range(8):
                wx = _bcast(wv, c * 8 + hloc)
                col = pl.ds(hloc * 16, 16)
                hhb[s2, j, col] = hhb[s2, j, col] * wx
            return cy

        lax.fori_loop(0, CHUNK, ebody, 0, unroll=4)

    row0 = s * CPT
    pltpu.sync_copy(src_h.at[row0], sbuf.at[0])
    pltpu.sync_copy(dst_h.at[row0], dbuf.at[0])
    ibuf_compute(0)
    gathers_issue(0, 0)
    idx_copy(1, 1)

    def chunk_iter(ci, carry):
        s0 = lax.rem(ci, 2)
        s1 = 1 - s0
        i0 = lax.rem(ci, 3)
        i1 = lax.rem(ci + 1, 3)
        i2 = lax.rem(ci + 2, 3)

        @pl.when(ci + 1 < CPT)
        def _():
            idx_wait(i1)
            ibuf_compute(i1)

            @pl.when(ci >= 1)
            def _():
                scatter_wait(s1, i2)

            gathers_issue(s1, i1)

        @pl.when(ci + 2 < CPT)
        def _():
            idx_copy(ci + 2, i2)

        gathers_wait(s0, i0)
        compute_chunk(s0)
        scatter_issue(s0, i0)
        return carry

    lax.fori_loop(0, CPT, chunk_iter, 0)
    scatter_wait((CPT - 2) % 2, (CPT - 2) % 3)
    scatter_wait((CPT - 1) % 2, (CPT - 1) % 3)
    plsc.subcore_barrier()
    pltpu.sync_copy(acc.at[pl.ds(rbase, RPT)], rstu_h.at[c, pl.ds(rbase, RPT)])


@functools.lru_cache(maxsize=1)
def _make_edge_pass():
    return functools.partial(
        pl.kernel,
        mesh=plsc.VectorSubcoreMesh(core_axis_name="c", subcore_axis_name="s"),
        compiler_params=pltpu.CompilerParams(
            use_tc_tiling_on_sc=False, needs_layout_passes=False),
        out_type=jax.ShapeDtypeStruct((2, NPAD, DW), jnp.float32),
        scratch_types=[
            pltpu.VMEM_SHARED((NPAD, DW), jnp.float32),
            pltpu.VMEM((3, CHUNK), jnp.int32),
            pltpu.VMEM((3, CHUNK), jnp.int32),
            pltpu.VMEM((3, CHUNK), jnp.int32),
            pltpu.VMEM((2, CHUNK, H), jnp.float32),
            pltpu.VMEM((2, CHUNK, DW), jnp.float32),
            pltpu.SemaphoreType.DMA((2,)),
            pltpu.SemaphoreType.DMA((3,)),
            pltpu.SemaphoreType.DMA((2,)),
        ],
    )(_edge_body)


def _edge_pass(*args):
    return _make_edge_pass()(*args)


def kernel(x, edge_index, emb_W1, emb_b1, emb_bn_g, emb_bn_b, emb_W2, emb_b2,
           gat_W, attn_l, attn_r, gat_bias, bn1_g, bn1_b, ff_W1, ff_b1,
           ff_W2, ff_b2, bn2_g, bn2_b, dec_W1, dec_b1, dec_bn_g, dec_bn_b,
           dec_W2, dec_b2):
    src = edge_index[0]
    dst = edge_index[1]
    src_p = src.reshape(CROWS, CHUNK)
    dst_p = dst.reshape(CROWS, CHUNK)
    r = lambda a: a.reshape(1, -1)

    eyeh = jnp.eye(H, dtype=jnp.float32)
    abig = (attn_l[:, :, :, None] * eyeh[None, :, None, :]).reshape(L, D, H)
    bbig = (attn_r[:, :, :, None] * eyeh[None, :, None, :]).reshape(L, D, H)
    e16 = jnp.kron(eyeh, jnp.ones((1, DH), jnp.float32))

    h = pl.pallas_call(_embed_body, out_shape=_f32((N, D)))(
        x, emb_W1, r(emb_b1), r(emb_bn_g), r(emb_bn_b), emb_W2, r(emb_b2))

    for l in range(L):
        hhx, er = pl.pallas_call(
            _attn_body, out_shape=[_f32((2, N, DW)), _f32((N, H))],
        )(h, gat_W[l], abig[l], bbig[l])

        hhx_cat = hhx.reshape(2 * N, DW)
        er_p = jnp.pad(er, ((0, NPAD - N), (0, 0)))
        rstu2 = _edge_pass(src_p, dst_p, hhx_cat, er_p)

        t = pl.pallas_call(
            _post1_body, out_shape=_f32((N, D)),
        )(h, rstu2[0, :N, :128], rstu2[1, :N, :128], rstu2[0, :N, 128:], e16,
          r(gat_bias[l]), r(bn1_g[l]), r(bn1_b[l]))
        h = pl.pallas_call(
            _post2_body, out_shape=_f32((N, D)),
        )(t, ff_W1[l], r(ff_b1[l]), ff_W2[l], r(ff_b2[l]), r(bn2_g[l]), r(bn2_b[l]))

    out = pl.pallas_call(_dec_body, out_shape=_f32((N, DOUT)))(
        h, dec_W1, r(dec_b1), r(dec_bn_g), r(dec_bn_b), dec_W2, r(dec_b2))
    return out
